# Initial kernel scaffold; baseline (speedup 1.0000x reference)
import functools

import jax
import jax.numpy as jnp
from jax import lax
from jax.experimental import pallas as pl
from jax.experimental.pallas import tpu as pltpu
from jax.experimental.pallas import tpu_sc as plsc

N = 10000
H = 122
DH = 128
NB = 16
RMAX = 0.3
STEPS = 3

NC, NS = 2, 16
NWK = NC * NS
GCH = 256
BE = 512
BN = 1000


def _sc_gather(x, snd_p, rcv_p):
    ep = snd_p.shape[0]
    ew = ep // NWK
    nch = ew // GCH
    mesh = plsc.VectorSubcoreMesh(core_axis_name="c", subcore_axis_name="s")

    @functools.partial(
        pl.kernel,
        out_type=[jax.ShapeDtypeStruct((ep, 128), jnp.float32),
                  jax.ShapeDtypeStruct((ep, 128), jnp.float32)],
        mesh=mesh,
        scratch_types=[pltpu.VMEM((GCH,), jnp.int32),
                       pltpu.VMEM((GCH, 128), jnp.float32),
                       pltpu.VMEM((GCH,), jnp.int32),
                       pltpu.VMEM((GCH, 128), jnp.float32),
                       pltpu.SemaphoreType.DMA,
                       pltpu.SemaphoreType.DMA],
    )
    def gk(x_hbm, snd_hbm, rcv_hbm, s_out, r_out,
           idx_s, rows_s, idx_r, rows_r, sem_s, sem_r):
        wid = lax.axis_index("s") * NC + lax.axis_index("c")
        base = wid * ew
        for j in range(nch):
            off = base + j * GCH
            pltpu.sync_copy(snd_hbm.at[pl.ds(off, GCH)], idx_s)
            pltpu.sync_copy(rcv_hbm.at[pl.ds(off, GCH)], idx_r)
            cs = pltpu.async_copy(x_hbm.at[idx_s], rows_s, sem_s)
            cr = pltpu.async_copy(x_hbm.at[idx_r], rows_r, sem_r)
            cs.wait()
            pltpu.sync_copy(rows_s, s_out.at[pl.ds(off, GCH)])
            cr.wait()
            pltpu.sync_copy(rows_r, r_out.at[pl.ds(off, GCH)])

    return gk(x, snd_p, rcv_p)


def _sc_scatter(xe, rcv3d, zeros_nx):
    ep = xe.shape[0]
    ew = ep // NWK
    nrow = ew // 128
    nslice = N // NS
    mesh = plsc.VectorSubcoreMesh(core_axis_name="c", subcore_axis_name="s")

    @functools.partial(
        pl.kernel,
        out_type=jax.ShapeDtypeStruct((NC, N, NB), jnp.float32),
        mesh=mesh,
        scratch_types=[pltpu.VMEM((nrow, 128), jnp.int32),
                       pltpu.VMEM((128, NB), jnp.float32),
                       pltpu.VMEM_SHARED((N, NB), jnp.float32),
                       pltpu.SemaphoreType.DMA],
    )
    def sk(x_hbm, idx_hbm, z_hbm, out_hbm, idx_v, dat_v, acc, sem):
        c = lax.axis_index("c")
        s = lax.axis_index("s")
        wid = s * NC + c
        pltpu.sync_copy(z_hbm.at[pl.ds(s * nslice, nslice)],
                        acc.at[pl.ds(s * nslice, nslice)])
        pltpu.sync_copy(idx_hbm.at[wid], idx_v)
        plsc.subcore_barrier()
        base = wid * ew
        for j in range(nrow):
            pltpu.sync_copy(x_hbm.at[pl.ds(base + j * 128, 128)], dat_v)
            pltpu.sync_copy(dat_v, acc.at[idx_v.at[j]], add=True)
        plsc.subcore_barrier()
        pltpu.sync_copy(acc.at[pl.ds(s * nslice, nslice)],
                        out_hbm.at[c, pl.ds(s * nslice, nslice)])

    return sk(xe, rcv3d, zeros_nx)


def _edge_body(s_ref, r_ref, wb_ref, wst_ref, bst_ref, xo_ref, *, e_real):
    i = pl.program_id(0)
    s = s_ref[...]
    r = r_ref[...]
    lane = lax.broadcasted_iota(jnp.int32, (BE, 128), 1)
    diff = jnp.where(lane < 3, s - r + 1e-7, 0.0)
    d2 = jnp.sum(diff * diff, axis=1, keepdims=True)
    d = jnp.maximum(jnp.sqrt(d2), 1e-7)
    k = (lax.broadcasted_iota(jnp.float32, (BE, NB), 1) + 1.0) * jnp.pi
    basis = jnp.sin(k * (d / RMAX)) * (jnp.sqrt(2.0 / RMAX) / d)
    m = basis @ wb_ref[...] + s @ wst_ref[0] + r @ wst_ref[1] + bst_ref[0][None, :]
    m = jax.nn.gelu(m)
    m = jax.nn.gelu(m @ wst_ref[2] + bst_ref[1][None, :])
    m = jax.nn.gelu(m @ wst_ref[3] + bst_ref[2][None, :])
    a = jax.nn.sigmoid(m @ wst_ref[4] + bst_ref[3][None, :])
    m = m * a
    t = jax.nn.gelu(m @ wst_ref[5] + bst_ref[4][None, :])
    t = jax.nn.gelu(t @ wst_ref[6] + bst_ref[5][None, :])
    trans = jnp.sum(t * bst_ref[6][None, :], axis=1, keepdims=True)
    xij = jnp.clip(diff * trans, -100.0, 100.0)
    row = i * BE + lax.broadcasted_iota(jnp.int32, (BE, NB), 0)
    xo_ref[...] = jnp.where(row < e_real, xij[:, :NB], 0.0)


def _edge_mlp(S, R, wb, wst, bst, e_real):
    ep = S.shape[0]
    return pl.pallas_call(
        functools.partial(_edge_body, e_real=e_real),
        grid=(ep // BE,),
        in_specs=[
            pl.BlockSpec((BE, 128), lambda i: (i, 0)),
            pl.BlockSpec((BE, 128), lambda i: (i, 0)),
            pl.BlockSpec((NB, 128), lambda i: (0, 0)),
            pl.BlockSpec((7, 128, 128), lambda i: (0, 0, 0)),
            pl.BlockSpec((8, 128), lambda i: (0, 0)),
        ],
        out_specs=pl.BlockSpec((BE, NB), lambda i: (i, 0)),
        out_shape=jax.ShapeDtypeStruct((ep, NB), jnp.float32),
    )(S, R, wb, wst, bst)


def _node_body(x_ref, xp_ref, nw_ref, nb_ref, wh2_ref, misc_ref, out_ref):
    xb = x_ref[...]
    sumx = xp_ref[0] + xp_ref[1]
    a = jax.nn.gelu(xb @ nw_ref[0] + nb_ref[0][None, :])
    a = jax.nn.gelu(a @ nw_ref[1] + nb_ref[1][None, :])
    vs = jnp.sum(a * misc_ref[0][None, :] + misc_ref[2][None, :],
                 axis=1, keepdims=True)
    b = jax.nn.gelu(xb @ nw_ref[2] + nb_ref[2][None, :])
    b = jax.nn.gelu(b @ nw_ref[3] + nb_ref[3][None, :])
    dh = b @ wh2_ref[...] + misc_ref[1][None, :]
    lane = lax.broadcasted_iota(jnp.int32, (BN, 128), 1)
    prow = lax.broadcasted_iota(jnp.int32, (128, 128), 0)
    pcol = lax.broadcasted_iota(jnp.int32, (128, 128), 1)
    pshift = jnp.where((prow == pcol + 3) & (pcol < 3), 1.0, 0.0)
    vn0 = xb @ pshift
    erow = lax.broadcasted_iota(jnp.int32, (NB, 128), 0)
    ecol = lax.broadcasted_iota(jnp.int32, (NB, 128), 1)
    emb = jnp.where(erow == ecol, 1.0, 0.0)
    sumx128 = sumx @ emb
    out_ref[...] = xb + jnp.where(
        lane < 3, vn0 * vs + sumx128,
        jnp.where(lane < 6, 0.0, dh))


def _node_update(x, xp, nw, nbias, wh2, misc):
    return pl.pallas_call(
        _node_body,
        grid=(N // BN,),
        in_specs=[
            pl.BlockSpec((BN, 128), lambda i: (i, 0)),
            pl.BlockSpec((2, BN, NB), lambda i: (0, i, 0)),
            pl.BlockSpec((4, 128, 128), lambda i: (0, 0, 0)),
            pl.BlockSpec((4, 128), lambda i: (0, 0)),
            pl.BlockSpec((128, 128), lambda i: (0, 0)),
            pl.BlockSpec((4, 128), lambda i: (0, 0)),
        ],
        out_specs=pl.BlockSpec((BN, 128), lambda i: (i, 0)),
        out_shape=jax.ShapeDtypeStruct((N, 128), jnp.float32),
    )(x, xp, nw, nbias, wh2, misc)


def _prep_step(p):
    z = jnp.zeros((128, 128), jnp.float32)
    pe, pa, px = p["phi_e"], p["phi_a"], p["phi_x"]
    w0 = pe[0]["W"]
    wb = w0[0:NB]
    w0s = z.at[6:128].set(w0[NB:NB + H])
    w0r = z.at[6:128].set(w0[NB + H:NB + 2 * H])
    wst = jnp.stack([w0s, w0r, pe[1]["W"], pe[2]["W"], pa[0]["W"],
                     px[0]["W"], px[1]["W"]])
    bst = jnp.stack([pe[0]["b"], pe[1]["b"], pe[2]["b"], pa[0]["b"],
                     px[0]["b"], px[1]["b"], p["phi_x_last"][:, 0],
                     jnp.zeros((128,), jnp.float32)])
    pv, ph = p["phi_v"], p["phi_h"]
    wv0 = z.at[6:128].set(pv[0]["W"])
    wh0 = z.at[6:128].set(ph[0]["W"])
    nw = jnp.stack([wv0, pv[1]["W"], wh0, ph[1]["W"]])
    nbias = jnp.stack([pv[0]["b"], pv[1]["b"], ph[0]["b"], ph[1]["b"]])
    wh2 = z.at[:, 6:128].set(ph[2]["W"])
    bh2 = jnp.zeros((128,), jnp.float32).at[6:128].set(ph[2]["b"])
    misc = jnp.stack([pv[2]["W"][:, 0],
                      bh2,
                      jnp.full((128,), pv[2]["b"][0] / 128.0, jnp.float32),
                      jnp.zeros((128,), jnp.float32)])
    return wb, wst, bst, nw, nbias, wh2, misc


def kernel(nodes, edge_index, params):
    e = edge_index.shape[1]
    ew = -(-e // (NWK * 512)) * 512
    ep = NWK * ew
    pad = ep - e
    snd_p = jnp.concatenate([edge_index[0], jnp.zeros((pad,), jnp.int32)])
    rcv_p = jnp.concatenate([edge_index[1], jnp.zeros((pad,), jnp.int32)])
    rcv3d = rcv_p.reshape(NWK, ew // 128, 128)
    zeros_nx = jnp.zeros((N, NB), jnp.float32)
    x = nodes
    for t in range(STEPS):
        wb, wst, bst, nw, nbias, wh2, misc = _prep_step(params["step%d" % t])
        S, R = _sc_gather(x, snd_p, rcv_p)
        xe = _edge_mlp(S, R, wb, wst, bst, e)
        xp = _sc_scatter(xe, rcv3d, zeros_nx)
        x = _node_update(x, xp, nw, nbias, wh2, misc)
    return x


# speedup vs baseline: 1.2445x; 1.2445x over previous
import functools

import jax
import jax.numpy as jnp
from jax import lax
from jax.experimental import pallas as pl
from jax.experimental.pallas import tpu as pltpu
from jax.experimental.pallas import tpu_sc as plsc

N = 10000
H = 122
DH = 128
NB = 16
RMAX = 0.3
STEPS = 3

NC, NS = 2, 16
NWK = NC * NS
GCH = 256
BE = 512
BN = 1000


def _sc_gather(x, snd_p, rcv_p):
    ep = snd_p.shape[0]
    ew = ep // NWK
    nch = ew // GCH
    mesh = plsc.VectorSubcoreMesh(core_axis_name="c", subcore_axis_name="s")

    @functools.partial(
        pl.kernel,
        out_type=[jax.ShapeDtypeStruct((ep, 128), jnp.float32),
                  jax.ShapeDtypeStruct((ep, 128), jnp.float32)],
        mesh=mesh,
        scratch_types=[pltpu.VMEM((GCH,), jnp.int32),
                       pltpu.VMEM((GCH, 128), jnp.float32),
                       pltpu.VMEM((GCH,), jnp.int32),
                       pltpu.VMEM((GCH, 128), jnp.float32),
                       pltpu.SemaphoreType.DMA,
                       pltpu.SemaphoreType.DMA],
    )
    def gk(x_hbm, snd_hbm, rcv_hbm, s_out, r_out,
           idx_s, rows_s, idx_r, rows_r, sem_s, sem_r):
        wid = lax.axis_index("s") * NC + lax.axis_index("c")
        base = wid * ew
        for j in range(nch):
            off = base + j * GCH
            pltpu.sync_copy(snd_hbm.at[pl.ds(off, GCH)], idx_s)
            pltpu.sync_copy(rcv_hbm.at[pl.ds(off, GCH)], idx_r)
            cs = pltpu.async_copy(x_hbm.at[idx_s], rows_s, sem_s)
            cr = pltpu.async_copy(x_hbm.at[idx_r], rows_r, sem_r)
            cs.wait()
            pltpu.sync_copy(rows_s, s_out.at[pl.ds(off, GCH)])
            cr.wait()
            pltpu.sync_copy(rows_r, r_out.at[pl.ds(off, GCH)])

    return gk(x, snd_p, rcv_p)


NP = 10240


def _sc_scatter(xe, rcv3d, zeros_nx):
    ep = xe.shape[0]
    ew = ep // NWK
    nrow = ew // 128
    nslice = NP // NS
    mesh = plsc.VectorSubcoreMesh(core_axis_name="c", subcore_axis_name="s")

    @functools.partial(
        pl.kernel,
        out_type=jax.ShapeDtypeStruct((NC, NP, 128), jnp.float32),
        mesh=mesh,
        scratch_types=[pltpu.VMEM((nrow, 128), jnp.int32),
                       pltpu.VMEM((128, 128), jnp.float32),
                       pltpu.VMEM_SHARED((NP, 128), jnp.float32),
                       pltpu.SemaphoreType.DMA],
    )
    def sk(x_hbm, idx_hbm, z_hbm, out_hbm, idx_v, dat_v, acc, sem):
        c = lax.axis_index("c")
        s = lax.axis_index("s")
        wid = s * NC + c
        pltpu.sync_copy(z_hbm.at[pl.ds(s * nslice, nslice)],
                        acc.at[pl.ds(s * nslice, nslice)])
        pltpu.sync_copy(idx_hbm.at[wid], idx_v)
        plsc.subcore_barrier()
        base = wid * ew
        for j in range(nrow):
            pltpu.sync_copy(x_hbm.at[pl.ds(base + j * 128, 128)], dat_v)
            pltpu.sync_copy(dat_v, acc.at[idx_v.at[j]], add=True)
        plsc.subcore_barrier()
        pltpu.sync_copy(acc.at[pl.ds(s * nslice, nslice)],
                        out_hbm.at[c, pl.ds(s * nslice, nslice)])

    return sk(xe, rcv3d, zeros_nx)


def _edge_body(s_ref, r_ref, wb_ref, wst_ref, bst_ref, xo_ref, *, e_real):
    i = pl.program_id(0)
    s = s_ref[...]
    r = r_ref[...]
    lane = lax.broadcasted_iota(jnp.int32, (BE, 128), 1)
    diff = jnp.where(lane < 3, s - r + 1e-7, 0.0)
    d2 = jnp.sum(diff * diff, axis=1, keepdims=True)
    d = jnp.maximum(jnp.sqrt(d2), 1e-7)
    k = (lax.broadcasted_iota(jnp.int32, (BE, NB), 1) + 1).astype(jnp.float32) * jnp.pi
    basis = jnp.sin(k * (d / RMAX)) * (jnp.sqrt(2.0 / RMAX) / d)
    m = basis @ wb_ref[...] + s @ wst_ref[0] + r @ wst_ref[1] + bst_ref[0][None, :]
    m = jax.nn.gelu(m)
    m = jax.nn.gelu(m @ wst_ref[2] + bst_ref[1][None, :])
    m = jax.nn.gelu(m @ wst_ref[3] + bst_ref[2][None, :])
    a = jax.nn.sigmoid(m @ wst_ref[4] + bst_ref[3][None, :])
    m = m * a
    t = jax.nn.gelu(m @ wst_ref[5] + bst_ref[4][None, :])
    t = jax.nn.gelu(t @ wst_ref[6] + bst_ref[5][None, :])
    trans = jnp.sum(t * bst_ref[6][None, :], axis=1, keepdims=True)
    xij = jnp.clip(diff * trans, -100.0, 100.0)
    row = i * BE + lax.broadcasted_iota(jnp.int32, (BE, 128), 0)
    xo_ref[...] = jnp.where(row < e_real, xij, 0.0)


def _edge_mlp(S, R, wb, wst, bst, e_real):
    ep = S.shape[0]
    return pl.pallas_call(
        functools.partial(_edge_body, e_real=e_real),
        grid=(ep // BE,),
        in_specs=[
            pl.BlockSpec((BE, 128), lambda i: (i, 0)),
            pl.BlockSpec((BE, 128), lambda i: (i, 0)),
            pl.BlockSpec((NB, 128), lambda i: (0, 0)),
            pl.BlockSpec((7, 128, 128), lambda i: (0, 0, 0)),
            pl.BlockSpec((8, 128), lambda i: (0, 0)),
        ],
        out_specs=pl.BlockSpec((BE, 128), lambda i: (i, 0)),
        out_shape=jax.ShapeDtypeStruct((ep, 128), jnp.float32),
    )(S, R, wb, wst, bst)


def _node_body(x_ref, xp_ref, nw_ref, nb_ref, wh2_ref, misc_ref, out_ref):
    xb = x_ref[...]
    sumx128 = xp_ref[0] + xp_ref[1]
    a = jax.nn.gelu(xb @ nw_ref[0] + nb_ref[0][None, :])
    a = jax.nn.gelu(a @ nw_ref[1] + nb_ref[1][None, :])
    vs = jnp.sum(a * misc_ref[0][None, :] + misc_ref[2][None, :],
                 axis=1, keepdims=True)
    b = jax.nn.gelu(xb @ nw_ref[2] + nb_ref[2][None, :])
    b = jax.nn.gelu(b @ nw_ref[3] + nb_ref[3][None, :])
    dh = b @ wh2_ref[...] + misc_ref[1][None, :]
    lane = lax.broadcasted_iota(jnp.int32, (BN, 128), 1)
    prow = lax.broadcasted_iota(jnp.int32, (128, 128), 0)
    pcol = lax.broadcasted_iota(jnp.int32, (128, 128), 1)
    pshift = jnp.where((prow == pcol + 3) & (pcol < 3), 1.0, 0.0)
    vn0 = xb @ pshift
    out_ref[...] = xb + jnp.where(
        lane < 3, vn0 * vs + sumx128,
        jnp.where(lane < 6, 0.0, dh))


def _node_update(x, xp, nw, nbias, wh2, misc):
    return pl.pallas_call(
        _node_body,
        grid=(N // BN,),
        in_specs=[
            pl.BlockSpec((BN, 128), lambda i: (i, 0)),
            pl.BlockSpec((2, BN, 128), lambda i: (0, i, 0)),
            pl.BlockSpec((4, 128, 128), lambda i: (0, 0, 0)),
            pl.BlockSpec((4, 128), lambda i: (0, 0)),
            pl.BlockSpec((128, 128), lambda i: (0, 0)),
            pl.BlockSpec((4, 128), lambda i: (0, 0)),
        ],
        out_specs=pl.BlockSpec((BN, 128), lambda i: (i, 0)),
        out_shape=jax.ShapeDtypeStruct((N, 128), jnp.float32),
    )(x, xp, nw, nbias, wh2, misc)


def _prep_step(p):
    z = jnp.zeros((128, 128), jnp.float32)
    pe, pa, px = p["phi_e"], p["phi_a"], p["phi_x"]
    w0 = pe[0]["W"]
    wb = w0[0:NB]
    w0s = z.at[6:128].set(w0[NB:NB + H])
    w0r = z.at[6:128].set(w0[NB + H:NB + 2 * H])
    wst = jnp.stack([w0s, w0r, pe[1]["W"], pe[2]["W"], pa[0]["W"],
                     px[0]["W"], px[1]["W"]])
    bst = jnp.stack([pe[0]["b"], pe[1]["b"], pe[2]["b"], pa[0]["b"],
                     px[0]["b"], px[1]["b"], p["phi_x_last"][:, 0],
                     jnp.zeros((128,), jnp.float32)])
    pv, ph = p["phi_v"], p["phi_h"]
    wv0 = z.at[6:128].set(pv[0]["W"])
    wh0 = z.at[6:128].set(ph[0]["W"])
    nw = jnp.stack([wv0, pv[1]["W"], wh0, ph[1]["W"]])
    nbias = jnp.stack([pv[0]["b"], pv[1]["b"], ph[0]["b"], ph[1]["b"]])
    wh2 = z.at[:, 6:128].set(ph[2]["W"])
    bh2 = jnp.zeros((128,), jnp.float32).at[6:128].set(ph[2]["b"])
    misc = jnp.stack([pv[2]["W"][:, 0],
                      bh2,
                      jnp.full((128,), pv[2]["b"][0] / 128.0, jnp.float32),
                      jnp.zeros((128,), jnp.float32)])
    return wb, wst, bst, nw, nbias, wh2, misc


def kernel(nodes, edge_index, params):
    e = edge_index.shape[1]
    ew = -(-e // (NWK * 512)) * 512
    ep = NWK * ew
    pad = ep - e
    snd_p = jnp.concatenate([edge_index[0], jnp.zeros((pad,), jnp.int32)])
    rcv_p = jnp.concatenate([edge_index[1], jnp.zeros((pad,), jnp.int32)])
    rcv3d = rcv_p.reshape(NWK, ew // 128, 128)
    zeros_nx = jnp.zeros((NP, 128), jnp.float32)
    x = nodes
    for t in range(STEPS):
        wb, wst, bst, nw, nbias, wh2, misc = _prep_step(params["step%d" % t])
        S, R = _sc_gather(x, snd_p, rcv_p)
        xe = _edge_mlp(S, R, wb, wst, bst, e)
        xp = _sc_scatter(xe, rcv3d, zeros_nx)
        x = _node_update(x, xp, nw, nbias, wh2, misc)
    return x


# speedup vs baseline: 1.3124x; 1.0546x over previous
import functools

import jax
import jax.numpy as jnp
from jax import lax
from jax.experimental import pallas as pl
from jax.experimental.pallas import tpu as pltpu
from jax.experimental.pallas import tpu_sc as plsc

N = 10000
H = 122
DH = 128
NB = 16
RMAX = 0.3
STEPS = 3

NC, NS = 2, 16
NWK = NC * NS
GCH = 160
BE = 512
BN = 1000


def _sc_gather(x, snd_p, rcv_p):
    ep = snd_p.shape[0]
    ew = ep // NWK
    nch = ew // GCH
    mesh = plsc.VectorSubcoreMesh(core_axis_name="c", subcore_axis_name="s")

    @functools.partial(
        pl.kernel,
        out_type=[jax.ShapeDtypeStruct((ep, 128), jnp.float32),
                  jax.ShapeDtypeStruct((ep, 128), jnp.float32)],
        mesh=mesh,
        scratch_types=[pltpu.VMEM((ew,), jnp.int32),
                       pltpu.VMEM((ew,), jnp.int32),
                       pltpu.VMEM((GCH, 128), jnp.float32),
                       pltpu.VMEM((GCH, 128), jnp.float32),
                       pltpu.VMEM((GCH, 128), jnp.float32),
                       pltpu.VMEM((GCH, 128), jnp.float32),
                       [pltpu.SemaphoreType.DMA] * 4,
                       [pltpu.SemaphoreType.DMA] * 4],
    )
    def gk(x_hbm, snd_hbm, rcv_hbm, s_out, r_out,
           idx_s, idx_r, buf_s0, buf_s1, buf_r0, buf_r1, gsems, wsems):
        wid = lax.axis_index("s") * NC + lax.axis_index("c")
        base = wid * ew
        pltpu.sync_copy(snd_hbm.at[pl.ds(base, ew)], idx_s)
        pltpu.sync_copy(rcv_hbm.at[pl.ds(base, ew)], idx_r)
        sb = [buf_s0, buf_s1]
        rb = [buf_r0, buf_r1]
        gd = [None, None]
        wd = [None, None]
        for j in range(nch + 1):
            b = j % 2
            if j < nch:
                if wd[b] is not None:
                    wd[b][0].wait()
                    wd[b][1].wait()
                    wd[b] = None
                io = pl.ds(j * GCH, GCH)
                gd[b] = (
                    pltpu.async_copy(x_hbm.at[idx_s.at[io]], sb[b],
                                     gsems[2 * b]),
                    pltpu.async_copy(x_hbm.at[idx_r.at[io]], rb[b],
                                     gsems[2 * b + 1]),
                )
            pb = (j + 1) % 2
            if j >= 1 and gd[pb] is not None:
                gd[pb][0].wait()
                gd[pb][1].wait()
                gd[pb] = None
                oo = pl.ds(base + (j - 1) * GCH, GCH)
                wd[pb] = (
                    pltpu.async_copy(sb[pb], s_out.at[oo],
                                     wsems[2 * pb]),
                    pltpu.async_copy(rb[pb], r_out.at[oo],
                                     wsems[2 * pb + 1]),
                )
        for b in range(2):
            if wd[b] is not None:
                wd[b][0].wait()
                wd[b][1].wait()

    return gk(x, snd_p, rcv_p)


NP = 10240


def _sc_scatter(xe, rcv3d, zeros_nx):
    ep = xe.shape[0]
    ew = ep // NWK
    nrow = ew // 128
    nslice = NP // NS
    mesh = plsc.VectorSubcoreMesh(core_axis_name="c", subcore_axis_name="s")

    @functools.partial(
        pl.kernel,
        out_type=jax.ShapeDtypeStruct((NC, NP, 128), jnp.float32),
        mesh=mesh,
        scratch_types=[pltpu.VMEM((nrow, 128), jnp.int32),
                       pltpu.VMEM((128, 128), jnp.float32),
                       pltpu.VMEM_SHARED((NP, 128), jnp.float32),
                       pltpu.SemaphoreType.DMA],
    )
    def sk(x_hbm, idx_hbm, z_hbm, out_hbm, idx_v, dat_v, acc, sem):
        c = lax.axis_index("c")
        s = lax.axis_index("s")
        wid = s * NC + c
        pltpu.sync_copy(z_hbm.at[pl.ds(s * nslice, nslice)],
                        acc.at[pl.ds(s * nslice, nslice)])
        pltpu.sync_copy(idx_hbm.at[wid], idx_v)
        plsc.subcore_barrier()
        base = wid * ew
        for j in range(nrow):
            pltpu.sync_copy(x_hbm.at[pl.ds(base + j * 128, 128)], dat_v)
            pltpu.sync_copy(dat_v, acc.at[idx_v.at[j]], add=True)
        plsc.subcore_barrier()
        pltpu.sync_copy(acc.at[pl.ds(s * nslice, nslice)],
                        out_hbm.at[c, pl.ds(s * nslice, nslice)])

    return sk(xe, rcv3d, zeros_nx)


def _edge_body(s_ref, r_ref, wb_ref, wst_ref, bst_ref, xo_ref, *, e_real):
    i = pl.program_id(0)
    s = s_ref[...]
    r = r_ref[...]
    lane = lax.broadcasted_iota(jnp.int32, (BE, 128), 1)
    diff = jnp.where(lane < 3, s - r + 1e-7, 0.0)
    d2 = jnp.sum(diff * diff, axis=1, keepdims=True)
    d = jnp.maximum(jnp.sqrt(d2), 1e-7)
    k = (lax.broadcasted_iota(jnp.int32, (BE, NB), 1) + 1).astype(jnp.float32) * jnp.pi
    basis = jnp.sin(k * (d / RMAX)) * (jnp.sqrt(2.0 / RMAX) / d)
    m = basis @ wb_ref[...] + s @ wst_ref[0] + r @ wst_ref[1] + bst_ref[0][None, :]
    m = jax.nn.gelu(m)
    m = jax.nn.gelu(m @ wst_ref[2] + bst_ref[1][None, :])
    m = jax.nn.gelu(m @ wst_ref[3] + bst_ref[2][None, :])
    a = jax.nn.sigmoid(m @ wst_ref[4] + bst_ref[3][None, :])
    m = m * a
    t = jax.nn.gelu(m @ wst_ref[5] + bst_ref[4][None, :])
    t = jax.nn.gelu(t @ wst_ref[6] + bst_ref[5][None, :])
    trans = jnp.sum(t * bst_ref[6][None, :], axis=1, keepdims=True)
    xij = jnp.clip(diff * trans, -100.0, 100.0)
    row = i * BE + lax.broadcasted_iota(jnp.int32, (BE, 128), 0)
    xo_ref[...] = jnp.where(row < e_real, xij, 0.0)


def _edge_mlp(S, R, wb, wst, bst, e_real):
    ep = S.shape[0]
    return pl.pallas_call(
        functools.partial(_edge_body, e_real=e_real),
        grid=(ep // BE,),
        in_specs=[
            pl.BlockSpec((BE, 128), lambda i: (i, 0)),
            pl.BlockSpec((BE, 128), lambda i: (i, 0)),
            pl.BlockSpec((NB, 128), lambda i: (0, 0)),
            pl.BlockSpec((7, 128, 128), lambda i: (0, 0, 0)),
            pl.BlockSpec((8, 128), lambda i: (0, 0)),
        ],
        out_specs=pl.BlockSpec((BE, 128), lambda i: (i, 0)),
        out_shape=jax.ShapeDtypeStruct((ep, 128), jnp.float32),
    )(S, R, wb, wst, bst)


def _node_body(x_ref, xp_ref, nw_ref, nb_ref, wh2_ref, misc_ref, out_ref):
    xb = x_ref[...]
    sumx128 = xp_ref[0] + xp_ref[1]
    a = jax.nn.gelu(xb @ nw_ref[0] + nb_ref[0][None, :])
    a = jax.nn.gelu(a @ nw_ref[1] + nb_ref[1][None, :])
    vs = jnp.sum(a * misc_ref[0][None, :] + misc_ref[2][None, :],
                 axis=1, keepdims=True)
    b = jax.nn.gelu(xb @ nw_ref[2] + nb_ref[2][None, :])
    b = jax.nn.gelu(b @ nw_ref[3] + nb_ref[3][None, :])
    dh = b @ wh2_ref[...] + misc_ref[1][None, :]
    lane = lax.broadcasted_iota(jnp.int32, (BN, 128), 1)
    prow = lax.broadcasted_iota(jnp.int32, (128, 128), 0)
    pcol = lax.broadcasted_iota(jnp.int32, (128, 128), 1)
    pshift = jnp.where((prow == pcol + 3) & (pcol < 3), 1.0, 0.0)
    vn0 = xb @ pshift
    out_ref[...] = xb + jnp.where(
        lane < 3, vn0 * vs + sumx128,
        jnp.where(lane < 6, 0.0, dh))


def _node_update(x, xp, nw, nbias, wh2, misc):
    return pl.pallas_call(
        _node_body,
        grid=(N // BN,),
        in_specs=[
            pl.BlockSpec((BN, 128), lambda i: (i, 0)),
            pl.BlockSpec((2, BN, 128), lambda i: (0, i, 0)),
            pl.BlockSpec((4, 128, 128), lambda i: (0, 0, 0)),
            pl.BlockSpec((4, 128), lambda i: (0, 0)),
            pl.BlockSpec((128, 128), lambda i: (0, 0)),
            pl.BlockSpec((4, 128), lambda i: (0, 0)),
        ],
        out_specs=pl.BlockSpec((BN, 128), lambda i: (i, 0)),
        out_shape=jax.ShapeDtypeStruct((N, 128), jnp.float32),
    )(x, xp, nw, nbias, wh2, misc)


def _prep_step(p):
    z = jnp.zeros((128, 128), jnp.float32)
    pe, pa, px = p["phi_e"], p["phi_a"], p["phi_x"]
    w0 = pe[0]["W"]
    wb = w0[0:NB]
    w0s = z.at[6:128].set(w0[NB:NB + H])
    w0r = z.at[6:128].set(w0[NB + H:NB + 2 * H])
    wst = jnp.stack([w0s, w0r, pe[1]["W"], pe[2]["W"], pa[0]["W"],
                     px[0]["W"], px[1]["W"]])
    bst = jnp.stack([pe[0]["b"], pe[1]["b"], pe[2]["b"], pa[0]["b"],
                     px[0]["b"], px[1]["b"], p["phi_x_last"][:, 0],
                     jnp.zeros((128,), jnp.float32)])
    pv, ph = p["phi_v"], p["phi_h"]
    wv0 = z.at[6:128].set(pv[0]["W"])
    wh0 = z.at[6:128].set(ph[0]["W"])
    nw = jnp.stack([wv0, pv[1]["W"], wh0, ph[1]["W"]])
    nbias = jnp.stack([pv[0]["b"], pv[1]["b"], ph[0]["b"], ph[1]["b"]])
    wh2 = z.at[:, 6:128].set(ph[2]["W"])
    bh2 = jnp.zeros((128,), jnp.float32).at[6:128].set(ph[2]["b"])
    misc = jnp.stack([pv[2]["W"][:, 0],
                      bh2,
                      jnp.full((128,), pv[2]["b"][0] / 128.0, jnp.float32),
                      jnp.zeros((128,), jnp.float32)])
    return wb, wst, bst, nw, nbias, wh2, misc


def kernel(nodes, edge_index, params):
    e = edge_index.shape[1]
    ew = -(-e // (NWK * 512)) * 512
    ep = NWK * ew
    pad = ep - e
    snd_p = jnp.concatenate([edge_index[0], jnp.zeros((pad,), jnp.int32)])
    rcv_p = jnp.concatenate([edge_index[1], jnp.zeros((pad,), jnp.int32)])
    rcv3d = rcv_p.reshape(NWK, ew // 128, 128)
    zeros_nx = jnp.zeros((NP, 128), jnp.float32)
    x = nodes
    for t in range(STEPS):
        wb, wst, bst, nw, nbias, wh2, misc = _prep_step(params["step%d" % t])
        S, R = _sc_gather(x, snd_p, rcv_p)
        xe = _edge_mlp(S, R, wb, wst, bst, e)
        xp = _sc_scatter(xe, rcv3d, zeros_nx)
        x = _node_update(x, xp, nw, nbias, wh2, misc)
    return x


# speedup vs baseline: 1.3799x; 1.0514x over previous
import functools

import jax
import jax.numpy as jnp
from jax import lax
from jax.experimental import pallas as pl
from jax.experimental.pallas import tpu as pltpu
from jax.experimental.pallas import tpu_sc as plsc

N = 10000
H = 122
DH = 128
NB = 16
RMAX = 0.3
STEPS = 3

NC, NS = 2, 16
NWK = NC * NS
GCH = 320
BE = 512
BN = 1000


def _sc_gather(x, ei2):
    ep = ei2.shape[1]
    ew = ep // NS
    nch = ew // GCH
    mesh = plsc.VectorSubcoreMesh(core_axis_name="c", subcore_axis_name="s")

    @functools.partial(
        pl.kernel,
        out_type=jax.ShapeDtypeStruct((2, ep, 128), jnp.float32),
        mesh=mesh,
        scratch_types=[pltpu.VMEM((ew,), jnp.int32),
                       pltpu.VMEM((GCH, 128), jnp.float32),
                       pltpu.VMEM((GCH, 128), jnp.float32),
                       [pltpu.SemaphoreType.DMA] * 2,
                       [pltpu.SemaphoreType.DMA] * 2],
    )
    def gk(x_hbm, ei_hbm, out_hbm, idx_v, b0, b1, gsems, wsems):
        c = lax.axis_index("c")
        s = lax.axis_index("s")
        base = s * ew
        pltpu.sync_copy(ei_hbm.at[c, pl.ds(base, ew)], idx_v)
        bufs = [b0, b1]
        gd = [None, None]
        wd = [None, None]
        for j in range(nch + 1):
            b = j % 2
            if j < nch:
                if wd[b] is not None:
                    wd[b].wait()
                    wd[b] = None
                gd[b] = pltpu.async_copy(
                    x_hbm.at[idx_v.at[pl.ds(j * GCH, GCH)]], bufs[b],
                    gsems[b])
            pb = (j + 1) % 2
            if j >= 1 and gd[pb] is not None:
                gd[pb].wait()
                gd[pb] = None
                wd[pb] = pltpu.async_copy(
                    bufs[pb], out_hbm.at[c, pl.ds(base + (j - 1) * GCH, GCH)],
                    wsems[pb])
        for b in range(2):
            if wd[b] is not None:
                wd[b].wait()

    return gk(x, ei2)


NP = 10240


def _sc_scatter(xe, rcv3d, zeros_nx):
    ep = xe.shape[0]
    ew = ep // NWK
    nrow = ew // 128
    nslice = NP // NS
    mesh = plsc.VectorSubcoreMesh(core_axis_name="c", subcore_axis_name="s")

    @functools.partial(
        pl.kernel,
        out_type=jax.ShapeDtypeStruct((NC, NP, 128), jnp.float32),
        mesh=mesh,
        scratch_types=[pltpu.VMEM((nrow, 128), jnp.int32),
                       pltpu.VMEM((128, 128), jnp.float32),
                       pltpu.VMEM_SHARED((NP, 128), jnp.float32),
                       pltpu.SemaphoreType.DMA],
    )
    def sk(x_hbm, idx_hbm, z_hbm, out_hbm, idx_v, dat_v, acc, sem):
        c = lax.axis_index("c")
        s = lax.axis_index("s")
        wid = s * NC + c
        pltpu.sync_copy(z_hbm.at[pl.ds(s * nslice, nslice)],
                        acc.at[pl.ds(s * nslice, nslice)])
        pltpu.sync_copy(idx_hbm.at[wid], idx_v)
        plsc.subcore_barrier()
        base = wid * ew
        for j in range(nrow):
            pltpu.sync_copy(x_hbm.at[pl.ds(base + j * 128, 128)], dat_v)
            pltpu.sync_copy(dat_v, acc.at[idx_v.at[j]], add=True)
        plsc.subcore_barrier()
        pltpu.sync_copy(acc.at[pl.ds(s * nslice, nslice)],
                        out_hbm.at[c, pl.ds(s * nslice, nslice)])

    return sk(xe, rcv3d, zeros_nx)


def _bdot(a, b):
    return jnp.dot(a.astype(jnp.bfloat16), b,
                   preferred_element_type=jnp.float32)


def _edge_body(sr_ref, wb_ref, wst_ref, bst_ref, xo_ref, *, e_real):
    i = pl.program_id(0)
    s = sr_ref[0]
    r = sr_ref[1]
    lane = lax.broadcasted_iota(jnp.int32, (BE, 128), 1)
    diff = jnp.where(lane < 3, s - r + 1e-7, 0.0)
    d2 = jnp.sum(diff * diff, axis=1, keepdims=True)
    d = jnp.maximum(jnp.sqrt(d2), 1e-7)
    k = (lax.broadcasted_iota(jnp.int32, (BE, NB), 1) + 1).astype(jnp.float32) * jnp.pi
    basis = jnp.sin(k * (d / RMAX)) * (jnp.sqrt(2.0 / RMAX) / d)
    m = (basis @ wb_ref[...] + _bdot(s, wst_ref[0]) + _bdot(r, wst_ref[1])
         + bst_ref[0][None, :])
    m = jax.nn.gelu(m)
    m = jax.nn.gelu(_bdot(m, wst_ref[2]) + bst_ref[1][None, :])
    m = jax.nn.gelu(_bdot(m, wst_ref[3]) + bst_ref[2][None, :])
    a = jax.nn.sigmoid(_bdot(m, wst_ref[4]) + bst_ref[3][None, :])
    m = m * a
    t = jax.nn.gelu(_bdot(m, wst_ref[5]) + bst_ref[4][None, :])
    t = jax.nn.gelu(_bdot(t, wst_ref[6]) + bst_ref[5][None, :])
    trans = jnp.sum(t * bst_ref[6][None, :], axis=1, keepdims=True)
    xij = jnp.clip(diff * trans, -100.0, 100.0)
    row = i * BE + lax.broadcasted_iota(jnp.int32, (BE, 128), 0)
    xo_ref[...] = jnp.where(row < e_real, xij, 0.0)


def _edge_mlp(SR, wb, wst, bst, e_real):
    ep = SR.shape[1]
    return pl.pallas_call(
        functools.partial(_edge_body, e_real=e_real),
        grid=(ep // BE,),
        in_specs=[
            pl.BlockSpec((2, BE, 128), lambda i: (0, i, 0)),
            pl.BlockSpec((NB, 128), lambda i: (0, 0)),
            pl.BlockSpec((7, 128, 128), lambda i: (0, 0, 0)),
            pl.BlockSpec((8, 128), lambda i: (0, 0)),
        ],
        out_specs=pl.BlockSpec((BE, 128), lambda i: (i, 0)),
        out_shape=jax.ShapeDtypeStruct((ep, 128), jnp.float32),
    )(SR, wb, wst, bst)


def _node_body(x_ref, xp_ref, nw_ref, nb_ref, wh2_ref, misc_ref, out_ref):
    xb = x_ref[...]
    sumx128 = xp_ref[0] + xp_ref[1]
    a = jax.nn.gelu(xb @ nw_ref[0] + nb_ref[0][None, :])
    a = jax.nn.gelu(a @ nw_ref[1] + nb_ref[1][None, :])
    vs = jnp.sum(a * misc_ref[0][None, :] + misc_ref[2][None, :],
                 axis=1, keepdims=True)
    b = jax.nn.gelu(xb @ nw_ref[2] + nb_ref[2][None, :])
    b = jax.nn.gelu(b @ nw_ref[3] + nb_ref[3][None, :])
    dh = b @ wh2_ref[...] + misc_ref[1][None, :]
    lane = lax.broadcasted_iota(jnp.int32, (BN, 128), 1)
    prow = lax.broadcasted_iota(jnp.int32, (128, 128), 0)
    pcol = lax.broadcasted_iota(jnp.int32, (128, 128), 1)
    pshift = jnp.where((prow == pcol + 3) & (pcol < 3), 1.0, 0.0)
    vn0 = xb @ pshift
    out_ref[...] = xb + jnp.where(
        lane < 3, vn0 * vs + sumx128,
        jnp.where(lane < 6, 0.0, dh))


def _node_update(x, xp, nw, nbias, wh2, misc):
    return pl.pallas_call(
        _node_body,
        grid=(N // BN,),
        in_specs=[
            pl.BlockSpec((BN, 128), lambda i: (i, 0)),
            pl.BlockSpec((2, BN, 128), lambda i: (0, i, 0)),
            pl.BlockSpec((4, 128, 128), lambda i: (0, 0, 0)),
            pl.BlockSpec((4, 128), lambda i: (0, 0)),
            pl.BlockSpec((128, 128), lambda i: (0, 0)),
            pl.BlockSpec((4, 128), lambda i: (0, 0)),
        ],
        out_specs=pl.BlockSpec((BN, 128), lambda i: (i, 0)),
        out_shape=jax.ShapeDtypeStruct((N, 128), jnp.float32),
    )(x, xp, nw, nbias, wh2, misc)


def _prep_step(p):
    z = jnp.zeros((128, 128), jnp.float32)
    pe, pa, px = p["phi_e"], p["phi_a"], p["phi_x"]
    w0 = pe[0]["W"]
    wb = w0[0:NB]
    w0s = z.at[6:128].set(w0[NB:NB + H])
    w0r = z.at[6:128].set(w0[NB + H:NB + 2 * H])
    wst = jnp.stack([w0s, w0r, pe[1]["W"], pe[2]["W"], pa[0]["W"],
                     px[0]["W"], px[1]["W"]]).astype(jnp.bfloat16)
    bst = jnp.stack([pe[0]["b"], pe[1]["b"], pe[2]["b"], pa[0]["b"],
                     px[0]["b"], px[1]["b"], p["phi_x_last"][:, 0],
                     jnp.zeros((128,), jnp.float32)])
    pv, ph = p["phi_v"], p["phi_h"]
    wv0 = z.at[6:128].set(pv[0]["W"])
    wh0 = z.at[6:128].set(ph[0]["W"])
    nw = jnp.stack([wv0, pv[1]["W"], wh0, ph[1]["W"]])
    nbias = jnp.stack([pv[0]["b"], pv[1]["b"], ph[0]["b"], ph[1]["b"]])
    wh2 = z.at[:, 6:128].set(ph[2]["W"])
    bh2 = jnp.zeros((128,), jnp.float32).at[6:128].set(ph[2]["b"])
    misc = jnp.stack([pv[2]["W"][:, 0],
                      bh2,
                      jnp.full((128,), pv[2]["b"][0] / 128.0, jnp.float32),
                      jnp.zeros((128,), jnp.float32)])
    return wb, wst, bst, nw, nbias, wh2, misc


def kernel(nodes, edge_index, params):
    e = edge_index.shape[1]
    ew = -(-e // (NWK * 512)) * 512
    ep = NWK * ew
    pad = ep - e
    ei2 = jnp.concatenate([edge_index, jnp.zeros((2, pad), jnp.int32)], axis=1)
    rcv3d = ei2[1].reshape(NWK, ew // 128, 128)
    zeros_nx = jnp.zeros((NP, 128), jnp.float32)
    x = nodes
    for t in range(STEPS):
        wb, wst, bst, nw, nbias, wh2, misc = _prep_step(params["step%d" % t])
        SR = _sc_gather(x, ei2)
        xe = _edge_mlp(SR, wb, wst, bst, e)
        xp = _sc_scatter(xe, rcv3d, zeros_nx)
        x = _node_update(x, xp, nw, nbias, wh2, misc)
    return x


# speedup vs baseline: 2.0391x; 1.4777x over previous
import functools

import jax
import jax.numpy as jnp
from jax import lax
from jax.experimental import pallas as pl
from jax.experimental.pallas import tpu as pltpu
from jax.experimental.pallas import tpu_sc as plsc

N = 10000
H = 122
DH = 128
NB = 16
RMAX = 0.3
STEPS = 3

NC, NS = 2, 16
NWK = NC * NS
GCH = 320
BE = 1024
BN = 1000


def _sc_gather(x, ei2):
    ep = ei2.shape[1]
    ew = ep // NS
    nch = ew // GCH
    mesh = plsc.VectorSubcoreMesh(core_axis_name="c", subcore_axis_name="s")

    @functools.partial(
        pl.kernel,
        out_type=jax.ShapeDtypeStruct((2, ep, 128), jnp.float32),
        mesh=mesh,
        scratch_types=[pltpu.VMEM((ew,), jnp.int32),
                       pltpu.VMEM((GCH, 128), jnp.float32),
                       pltpu.VMEM((GCH, 128), jnp.float32),
                       [pltpu.SemaphoreType.DMA] * 2,
                       [pltpu.SemaphoreType.DMA] * 2],
    )
    def gk(x_hbm, ei_hbm, out_hbm, idx_v, b0, b1, gsems, wsems):
        c = lax.axis_index("c")
        s = lax.axis_index("s")
        base = s * ew
        pltpu.sync_copy(ei_hbm.at[c, pl.ds(base, ew)], idx_v)
        bufs = [b0, b1]
        gd = [None, None]
        wd = [None, None]
        for j in range(nch + 1):
            b = j % 2
            if j < nch:
                if wd[b] is not None:
                    wd[b].wait()
                    wd[b] = None
                gd[b] = pltpu.async_copy(
                    x_hbm.at[idx_v.at[pl.ds(j * GCH, GCH)]], bufs[b],
                    gsems[b])
            pb = (j + 1) % 2
            if j >= 1 and gd[pb] is not None:
                gd[pb].wait()
                gd[pb] = None
                wd[pb] = pltpu.async_copy(
                    bufs[pb], out_hbm.at[c, pl.ds(base + (j - 1) * GCH, GCH)],
                    wsems[pb])
        for b in range(2):
            if wd[b] is not None:
                wd[b].wait()

    return gk(x, ei2)


NP = 10240


def _sc_scatter(xe, rcv3d, zeros_nx):
    ep = xe.shape[0]
    ew = ep // NWK
    nrow = ew // 128
    nslice = NP // NS
    mesh = plsc.VectorSubcoreMesh(core_axis_name="c", subcore_axis_name="s")

    @functools.partial(
        pl.kernel,
        out_type=jax.ShapeDtypeStruct((NC, NP, 128), jnp.float32),
        mesh=mesh,
        scratch_types=[pltpu.VMEM((nrow, 128), jnp.int32),
                       pltpu.VMEM((128, 128), jnp.float32),
                       pltpu.VMEM_SHARED((NP, 128), jnp.float32),
                       pltpu.SemaphoreType.DMA],
    )
    def sk(x_hbm, idx_hbm, z_hbm, out_hbm, idx_v, dat_v, acc, sem):
        c = lax.axis_index("c")
        s = lax.axis_index("s")
        wid = s * NC + c
        pltpu.sync_copy(z_hbm.at[pl.ds(s * nslice, nslice)],
                        acc.at[pl.ds(s * nslice, nslice)])
        pltpu.sync_copy(idx_hbm.at[wid], idx_v)
        plsc.subcore_barrier()
        base = wid * ew
        for j in range(nrow):
            pltpu.sync_copy(x_hbm.at[pl.ds(base + j * 128, 128)], dat_v)
            pltpu.sync_copy(dat_v, acc.at[idx_v.at[j]], add=True)
        plsc.subcore_barrier()
        pltpu.sync_copy(acc.at[pl.ds(s * nslice, nslice)],
                        out_hbm.at[c, pl.ds(s * nslice, nslice)])

    return sk(xe, rcv3d, zeros_nx)


def _fdot(a, b):
    return jnp.dot(a, b, preferred_element_type=jnp.float32)


_SINCOEF = (6.2830885, -41.3332475, 81.4000898, -74.6758839, 33.1680946)


def _sin2pi(u):
    u2 = u * u
    p = jnp.float32(_SINCOEF[-1])
    for c in _SINCOEF[-2::-1]:
        p = p * u2 + jnp.float32(c)
    return u * p


def _edge_body(sr_ref, wb_ref, wst_ref, bst_ref, xo_ref, *, e_real):
    i = pl.program_id(0)
    s = sr_ref[0]
    r = sr_ref[1]
    lane = lax.broadcasted_iota(jnp.int32, (BE, 128), 1)
    diff = jnp.where(lane < 3, s - r + 1e-7, 0.0)
    d2 = jnp.sum(diff * diff, axis=1, keepdims=True)
    d = jnp.maximum(jnp.sqrt(d2), 1e-7)
    k = (lax.broadcasted_iota(jnp.int32, (BE, NB), 1) + 1).astype(jnp.float32)
    t = k * (d * (0.5 / RMAX))
    u = t - jnp.round(t)
    basis = _sin2pi(u) * (jnp.sqrt(2.0 / RMAX) / d)
    bf = jnp.bfloat16
    m = (basis @ wb_ref[...] + _fdot(s.astype(bf), wst_ref[0])
         + _fdot(r.astype(bf), wst_ref[1]) + bst_ref[0][None, :])
    m = jax.nn.gelu(m.astype(bf))
    m = jax.nn.gelu((_fdot(m, wst_ref[2]) + bst_ref[1][None, :]).astype(bf))
    m = jax.nn.gelu((_fdot(m, wst_ref[3]) + bst_ref[2][None, :]).astype(bf))
    a = jax.nn.sigmoid((_fdot(m, wst_ref[4]) + bst_ref[3][None, :]).astype(bf))
    m = m * a
    t2 = jax.nn.gelu((_fdot(m, wst_ref[5]) + bst_ref[4][None, :]).astype(bf))
    t2 = jax.nn.gelu((_fdot(t2, wst_ref[6]) + bst_ref[5][None, :]).astype(bf))
    trans = jnp.sum(t2.astype(jnp.float32) * bst_ref[6][None, :],
                    axis=1, keepdims=True)
    xij = jnp.clip(diff * trans, -100.0, 100.0)
    row = i * BE + lax.broadcasted_iota(jnp.int32, (BE, 128), 0)
    xo_ref[...] = jnp.where(row < e_real, xij, 0.0)


def _edge_mlp(SR, wb, wst, bst, e_real):
    ep = SR.shape[1]
    return pl.pallas_call(
        functools.partial(_edge_body, e_real=e_real),
        grid=(ep // BE,),
        in_specs=[
            pl.BlockSpec((2, BE, 128), lambda i: (0, i, 0)),
            pl.BlockSpec((NB, 128), lambda i: (0, 0)),
            pl.BlockSpec((7, 128, 128), lambda i: (0, 0, 0)),
            pl.BlockSpec((8, 128), lambda i: (0, 0)),
        ],
        out_specs=pl.BlockSpec((BE, 128), lambda i: (i, 0)),
        out_shape=jax.ShapeDtypeStruct((ep, 128), jnp.float32),
    )(SR, wb, wst, bst)


def _node_body(x_ref, xp_ref, nw_ref, nb_ref, wh2_ref, misc_ref, out_ref):
    xb = x_ref[...]
    sumx128 = xp_ref[0] + xp_ref[1]
    a = jax.nn.gelu(xb @ nw_ref[0] + nb_ref[0][None, :])
    a = jax.nn.gelu(a @ nw_ref[1] + nb_ref[1][None, :])
    vs = jnp.sum(a * misc_ref[0][None, :] + misc_ref[2][None, :],
                 axis=1, keepdims=True)
    b = jax.nn.gelu(xb @ nw_ref[2] + nb_ref[2][None, :])
    b = jax.nn.gelu(b @ nw_ref[3] + nb_ref[3][None, :])
    dh = b @ wh2_ref[...] + misc_ref[1][None, :]
    lane = lax.broadcasted_iota(jnp.int32, (BN, 128), 1)
    prow = lax.broadcasted_iota(jnp.int32, (128, 128), 0)
    pcol = lax.broadcasted_iota(jnp.int32, (128, 128), 1)
    pshift = jnp.where((prow == pcol + 3) & (pcol < 3), 1.0, 0.0)
    vn0 = xb @ pshift
    out_ref[...] = xb + jnp.where(
        lane < 3, vn0 * vs + sumx128,
        jnp.where(lane < 6, 0.0, dh))


def _node_update(x, xp, nw, nbias, wh2, misc):
    return pl.pallas_call(
        _node_body,
        grid=(N // BN,),
        in_specs=[
            pl.BlockSpec((BN, 128), lambda i: (i, 0)),
            pl.BlockSpec((2, BN, 128), lambda i: (0, i, 0)),
            pl.BlockSpec((4, 128, 128), lambda i: (0, 0, 0)),
            pl.BlockSpec((4, 128), lambda i: (0, 0)),
            pl.BlockSpec((128, 128), lambda i: (0, 0)),
            pl.BlockSpec((4, 128), lambda i: (0, 0)),
        ],
        out_specs=pl.BlockSpec((BN, 128), lambda i: (i, 0)),
        out_shape=jax.ShapeDtypeStruct((N, 128), jnp.float32),
    )(x, xp, nw, nbias, wh2, misc)


def _prep_step(p):
    z = jnp.zeros((128, 128), jnp.float32)
    pe, pa, px = p["phi_e"], p["phi_a"], p["phi_x"]
    w0 = pe[0]["W"]
    wb = w0[0:NB]
    w0s = z.at[6:128].set(w0[NB:NB + H])
    w0r = z.at[6:128].set(w0[NB + H:NB + 2 * H])
    wst = jnp.stack([w0s, w0r, pe[1]["W"], pe[2]["W"], pa[0]["W"],
                     px[0]["W"], px[1]["W"]]).astype(jnp.bfloat16)
    bst = jnp.stack([pe[0]["b"], pe[1]["b"], pe[2]["b"], pa[0]["b"],
                     px[0]["b"], px[1]["b"], p["phi_x_last"][:, 0],
                     jnp.zeros((128,), jnp.float32)])
    pv, ph = p["phi_v"], p["phi_h"]
    wv0 = z.at[6:128].set(pv[0]["W"])
    wh0 = z.at[6:128].set(ph[0]["W"])
    nw = jnp.stack([wv0, pv[1]["W"], wh0, ph[1]["W"]])
    nbias = jnp.stack([pv[0]["b"], pv[1]["b"], ph[0]["b"], ph[1]["b"]])
    wh2 = z.at[:, 6:128].set(ph[2]["W"])
    bh2 = jnp.zeros((128,), jnp.float32).at[6:128].set(ph[2]["b"])
    misc = jnp.stack([pv[2]["W"][:, 0],
                      bh2,
                      jnp.full((128,), pv[2]["b"][0] / 128.0, jnp.float32),
                      jnp.zeros((128,), jnp.float32)])
    return wb, wst, bst, nw, nbias, wh2, misc


def kernel(nodes, edge_index, params):
    e = edge_index.shape[1]
    ew = -(-e // (NWK * 512)) * 512
    ep = NWK * ew
    pad = ep - e
    ei2 = jnp.concatenate([edge_index, jnp.zeros((2, pad), jnp.int32)], axis=1)
    rcv3d = ei2[1].reshape(NWK, ew // 128, 128)
    zeros_nx = jnp.zeros((NP, 128), jnp.float32)
    x = nodes
    for t in range(STEPS):
        wb, wst, bst, nw, nbias, wh2, misc = _prep_step(params["step%d" % t])
        SR = _sc_gather(x, ei2)
        xe = _edge_mlp(SR, wb, wst, bst, e)
        xp = _sc_scatter(xe, rcv3d, zeros_nx)
        x = _node_update(x, xp, nw, nbias, wh2, misc)
    return x
